# baseline (speedup 1.0000x reference)
import functools

import jax
import jax.numpy as jnp
from jax import lax
from jax.experimental import pallas as pl
from jax.experimental.pallas import tpu as pltpu
from jax.experimental.pallas import tpu_sc as plsc

NM = 8
BT = 4096
T = 256
PAD = BT + NM * T
NB = PAD // T
CHUNK = 512
NW = 32
RPW = BT // NW

def _sc_mesh():
    return plsc.VectorSubcoreMesh(core_axis_name="c", subcore_axis_name="s")


def _route_body(in_ref, dest0_ref, dest1_ref, gid0_ref, gid1_ref):
    oh = in_ref[:, 256:272]
    counts = jnp.sum(oh, axis=0, keepdims=True)
    padded = jnp.ceil(counts / T) * T
    ii = lax.broadcasted_iota(jnp.int32, (16, 16), 0)
    jj = lax.broadcasted_iota(jnp.int32, (16, 16), 1)
    excl = jnp.where((ii < jj) & ((ii >= 8) == (jj >= 8)), 1.0, 0.0)
    pad_off = jnp.dot(padded, excl, preferred_element_type=jnp.float32)
    pad_end = pad_off + padded

    ri = lax.broadcasted_iota(jnp.int32, (CHUNK, CHUNK), 0)
    rj = lax.broadcasted_iota(jnp.int32, (CHUNK, CHUNK), 1)
    tri = jnp.where(ri > rj, 1.0, 0.0)

    def body(k, carry):
        blk = in_ref[pl.ds(k * CHUNK, CHUNK), 256:272]
        ranks = jnp.dot(tri, blk, preferred_element_type=jnp.float32) + carry
        val = blk * (pad_off + ranks)
        d0 = jnp.sum(val[:, :8], axis=1, keepdims=True)
        d1 = jnp.sum(val[:, 8:], axis=1, keepdims=True)
        dest0_ref[pl.ds(k * CHUNK, CHUNK), :] = d0.astype(jnp.int32)
        dest1_ref[pl.ds(k * CHUNK, CHUNK), :] = d1.astype(jnp.int32)
        return carry + jnp.sum(blk, axis=0, keepdims=True)

    lax.fori_loop(0, BT // CHUNK, body, jnp.zeros((1, 16), jnp.float32))

    bstart = lax.broadcasted_iota(jnp.int32, (NB, 1), 0).astype(jnp.float32) * T
    ge0 = (bstart >= pad_end[:, :8]).astype(jnp.int32)
    ge1 = (bstart >= pad_end[:, 8:]).astype(jnp.int32)
    gid0_ref[...] = jnp.minimum(jnp.sum(ge0, axis=1, keepdims=True), NM - 1)
    gid1_ref[...] = jnp.minimum(jnp.sum(ge1, axis=1, keepdims=True), NM - 1)


def _route(input_val):
    out_shape = (
        jax.ShapeDtypeStruct((BT, 1), jnp.int32),
        jax.ShapeDtypeStruct((BT, 1), jnp.int32),
        jax.ShapeDtypeStruct((NB, 1), jnp.int32),
        jax.ShapeDtypeStruct((NB, 1), jnp.int32),
    )
    return pl.pallas_call(_route_body, out_shape=out_shape)(input_val)


def _sc_scatter_feats(input_val, dest0, dest1):
    @functools.partial(
        pl.kernel,
        out_type=(
            jax.ShapeDtypeStruct((PAD, 128), jnp.float32),
            jax.ShapeDtypeStruct((PAD, 128), jnp.float32),
        ),
        mesh=_sc_mesh(),
        scratch_types=[
            pltpu.VMEM((RPW,), jnp.int32),
            pltpu.VMEM((RPW,), jnp.int32),
            pltpu.VMEM((RPW, 128), jnp.float32),
            pltpu.VMEM((RPW, 128), jnp.float32),
        ],
    )
    def k(in_hbm, i0_hbm, i1_hbm, xs0_hbm, f1s_hbm, i0_v, i1_v, r0_v, r1_v):
        wid = lax.axis_index("s") * 2 + lax.axis_index("c")
        base = wid * RPW
        pltpu.sync_copy(i0_hbm.at[pl.ds(base, RPW)], i0_v)
        pltpu.sync_copy(i1_hbm.at[pl.ds(base, RPW)], i1_v)
        pltpu.sync_copy(in_hbm.at[pl.ds(base, RPW), pl.ds(0, 128)], r0_v)
        pltpu.sync_copy(in_hbm.at[pl.ds(base, RPW), pl.ds(128, 128)], r1_v)
        pltpu.sync_copy(r0_v, xs0_hbm.at[i0_v])
        pltpu.sync_copy(r1_v, f1s_hbm.at[i1_v])

    return k(input_val, dest0, dest1)


def _sc_transfer(x0s, dest0, dest1):
    c = 32

    @functools.partial(
        pl.kernel,
        out_type=jax.ShapeDtypeStruct((PAD, 512), jnp.int32),
        mesh=_sc_mesh(),
        scratch_types=[
            pltpu.VMEM((c,), jnp.int32),
            pltpu.VMEM((c,), jnp.int32),
            pltpu.VMEM((c, 512), jnp.int32),
        ],
    )
    def k(x_hbm, i0_hbm, i1_hbm, out_hbm, i0_v, i1_v, rows_v):
        wid = lax.axis_index("s") * 2 + lax.axis_index("c")
        base = wid * RPW

        @pl.loop(0, RPW, step=c)
        def _(off):
            pltpu.sync_copy(i0_hbm.at[pl.ds(base + off, c)], i0_v)
            pltpu.sync_copy(i1_hbm.at[pl.ds(base + off, c)], i1_v)
            pltpu.sync_copy(x_hbm.at[i0_v], rows_v)
            pltpu.sync_copy(rows_v, out_hbm.at[i1_v])

    return k(x0s, dest0, dest1)


def _sc_gather_out(os_, dest1):
    @functools.partial(
        pl.kernel,
        out_type=jax.ShapeDtypeStruct((BT, 512), jnp.float32),
        mesh=_sc_mesh(),
        scratch_types=[
            pltpu.VMEM((RPW,), jnp.int32),
            pltpu.VMEM((RPW, 512), jnp.float32),
        ],
    )
    def k(x_hbm, idx_hbm, out_hbm, idx_v, rows_v):
        wid = lax.axis_index("s") * 2 + lax.axis_index("c")
        base = wid * RPW
        pltpu.sync_copy(idx_hbm.at[pl.ds(base, RPW)], idx_v)
        pltpu.sync_copy(x_hbm.at[idx_v], rows_v)
        pltpu.sync_copy(rows_v, out_hbm.at[pl.ds(base, RPW)])

    return k(os_, dest1)


def _gemm_a_body(gid_ref, x_ref, w1_ref, b1_ref, w2_ref, b2_ref, o_ref):
    g = gid_ref[pl.program_id(0)]
    x = x_ref[...]
    h = jnp.dot(x, w1_ref[g], preferred_element_type=jnp.float32)
    h = jnp.maximum(h + b1_ref[g], 0.0)
    o = jnp.dot(h, w2_ref[g], preferred_element_type=jnp.float32)
    o_ref[...] = jnp.maximum(o + b2_ref[g], 0.0).astype(jnp.bfloat16)


def _gemm_a(gid, xs, w1, b1, w2, b2):
    grid_spec = pltpu.PrefetchScalarGridSpec(
        num_scalar_prefetch=1,
        grid=(NB,),
        in_specs=[
            pl.BlockSpec((T, 128), lambda b, g: (b, 0)),
            pl.BlockSpec((NM, 128, 1024), lambda b, g: (0, 0, 0)),
            pl.BlockSpec((NM, 1024), lambda b, g: (0, 0)),
            pl.BlockSpec((NM, 1024, 1024), lambda b, g: (0, 0, 0)),
            pl.BlockSpec((NM, 1024), lambda b, g: (0, 0)),
        ],
        out_specs=pl.BlockSpec((T, 1024), lambda b, g: (b, 0)),
    )
    return pl.pallas_call(
        _gemm_a_body,
        grid_spec=grid_spec,
        out_shape=jax.ShapeDtypeStruct((PAD, 1024), jnp.bfloat16),
    )(gid, xs, w1, b1, w2, b2)


def _gemm_b_body(gid_ref, f1_ref, x0_ref, pw_ref, pb_ref, w1_ref, b1_ref,
                 w2_ref, b2_ref, o_ref):
    g = gid_ref[pl.program_id(0)]
    p = jnp.dot(f1_ref[...], pw_ref[g], preferred_element_type=jnp.float32)
    p = jnp.maximum(p + pb_ref[g], 0.0)
    w1 = w1_ref[0]
    x0 = x0_ref[...].astype(jnp.float32)
    h = jnp.dot(x0, w1[:1024], preferred_element_type=jnp.float32)
    h = h + jnp.dot(p, w1[1024:], preferred_element_type=jnp.float32)
    h = jnp.maximum(h + b1_ref[g], 0.0)
    o = jnp.dot(h, w2_ref[g], preferred_element_type=jnp.float32)
    o_ref[...] = o + b2_ref[g]


def _gemm_b(gid, f1s, x0g, pw, pb, w1, b1, w2, b2):
    grid_spec = pltpu.PrefetchScalarGridSpec(
        num_scalar_prefetch=1,
        grid=(NB,),
        in_specs=[
            pl.BlockSpec((T, 128), lambda b, g: (b, 0)),
            pl.BlockSpec((T, 1024), lambda b, g: (b, 0)),
            pl.BlockSpec((NM, 128, 512), lambda b, g: (0, 0, 0)),
            pl.BlockSpec((NM, 512), lambda b, g: (0, 0)),
            pl.BlockSpec((1, 1536, 1024), lambda b, g: (g[b], 0, 0)),
            pl.BlockSpec((NM, 1024), lambda b, g: (0, 0)),
            pl.BlockSpec((NM, 1024, 512), lambda b, g: (0, 0, 0)),
            pl.BlockSpec((NM, 512), lambda b, g: (0, 0)),
        ],
        out_specs=pl.BlockSpec((T, 512), lambda b, g: (b, 0)),
    )
    return pl.pallas_call(
        _gemm_b_body,
        grid_spec=grid_spec,
        out_shape=jax.ShapeDtypeStruct((PAD, 512), jnp.float32),
    )(gid, f1s, x0g, pw, pb, w1, b1, w2, b2)


def kernel(input_val, n0_W1, n0_b1, n0_W2, n0_b2, n1_preW, n1_preb,
           n1_W1, n1_b1, n1_W2, n1_b2):
    dest0c, dest1c, gid0c, gid1c = _route(input_val)
    dest0 = dest0c.reshape(BT)
    dest1 = dest1c.reshape(BT)
    gid0 = gid0c.reshape(NB)
    gid1 = gid1c.reshape(NB)

    xs0, f1s = _sc_scatter_feats(input_val, dest0, dest1)
    x0s = _gemm_a(gid0, xs0, n0_W1, n0_b1, n0_W2, n0_b2)
    x0i = lax.bitcast_convert_type(x0s.reshape(PAD, 512, 2), jnp.int32)
    x0g = lax.bitcast_convert_type(_sc_transfer(x0i, dest0, dest1), jnp.bfloat16)
    x0g = x0g.reshape(PAD, 1024)
    os_ = _gemm_b(gid1, f1s, x0g, n1_preW, n1_preb, n1_W1, n1_b1, n1_W2, n1_b2)
    return _sc_gather_out(os_, dest1)


# speedup vs baseline: 2.7632x; 2.7632x over previous
import functools

import jax
import jax.numpy as jnp
from jax import lax
from jax.experimental import pallas as pl
from jax.experimental.pallas import tpu as pltpu
from jax.experimental.pallas import tpu_sc as plsc

NM = 8
BT = 4096
T = 256
PAD = BT + NM * T
NB = PAD // T
CHUNK = 512
NW = 32
RPW = BT // NW

def _sc_mesh():
    return plsc.VectorSubcoreMesh(core_axis_name="c", subcore_axis_name="s")


def _route_body(oh_ref, dest0_ref, dest1_ref, gid0_ref, gid1_ref):
    oh = oh_ref[...]
    counts = jnp.sum(oh, axis=0, keepdims=True)
    padded = jnp.ceil(counts / T) * T
    ii = lax.broadcasted_iota(jnp.int32, (16, 16), 0)
    jj = lax.broadcasted_iota(jnp.int32, (16, 16), 1)
    excl = jnp.where((ii < jj) & ((ii >= 8) == (jj >= 8)), 1.0, 0.0)
    pad_off = jnp.dot(padded, excl, preferred_element_type=jnp.float32)
    pad_end = pad_off + padded

    ri = lax.broadcasted_iota(jnp.int32, (CHUNK, CHUNK), 0)
    rj = lax.broadcasted_iota(jnp.int32, (CHUNK, CHUNK), 1)
    tri = jnp.where(ri > rj, 1.0, 0.0)

    def body(k, carry):
        blk = oh_ref[pl.ds(k * CHUNK, CHUNK), :]
        ranks = jnp.dot(tri, blk, preferred_element_type=jnp.float32) + carry
        val = blk * (pad_off + ranks)
        d0 = jnp.sum(val[:, :8], axis=1, keepdims=True)
        d1 = jnp.sum(val[:, 8:], axis=1, keepdims=True)
        dest0_ref[pl.ds(k * CHUNK, CHUNK), :] = d0.astype(jnp.int32)
        dest1_ref[pl.ds(k * CHUNK, CHUNK), :] = d1.astype(jnp.int32)
        return carry + jnp.sum(blk, axis=0, keepdims=True)

    lax.fori_loop(0, BT // CHUNK, body, jnp.zeros((1, 16), jnp.float32))

    bstart = lax.broadcasted_iota(jnp.int32, (NB, 1), 0).astype(jnp.float32) * T
    ge0 = (bstart >= pad_end[:, :8]).astype(jnp.int32)
    ge1 = (bstart >= pad_end[:, 8:]).astype(jnp.int32)
    gid0_ref[...] = jnp.minimum(jnp.sum(ge0, axis=1, keepdims=True), NM - 1)
    gid1_ref[...] = jnp.minimum(jnp.sum(ge1, axis=1, keepdims=True), NM - 1)


def _route(oh):
    out_shape = (
        jax.ShapeDtypeStruct((BT, 1), jnp.int32),
        jax.ShapeDtypeStruct((BT, 1), jnp.int32),
        jax.ShapeDtypeStruct((NB, 1), jnp.int32),
        jax.ShapeDtypeStruct((NB, 1), jnp.int32),
    )
    return pl.pallas_call(_route_body, out_shape=out_shape)(oh)


def _sc_scatter_rows(rows, dest, d):
    @functools.partial(
        pl.kernel,
        out_type=jax.ShapeDtypeStruct((PAD, d), jnp.float32),
        mesh=_sc_mesh(),
        scratch_types=[
            pltpu.VMEM((RPW,), jnp.int32),
            pltpu.VMEM((RPW, d), jnp.float32),
        ],
    )
    def k(rows_hbm, idx_hbm, out_hbm, idx_v, rows_v):
        wid = lax.axis_index("s") * 2 + lax.axis_index("c")
        base = wid * RPW
        pltpu.sync_copy(idx_hbm.at[pl.ds(base, RPW)], idx_v)
        pltpu.sync_copy(rows_hbm.at[pl.ds(base, RPW)], rows_v)
        pltpu.sync_copy(rows_v, out_hbm.at[idx_v])

    return k(rows, dest)


def _sc_transfer(x0s, dest0, dest1):
    c = 32

    @functools.partial(
        pl.kernel,
        out_type=jax.ShapeDtypeStruct((PAD, 1024), jnp.float32),
        mesh=_sc_mesh(),
        scratch_types=[
            pltpu.VMEM((c,), jnp.int32),
            pltpu.VMEM((c,), jnp.int32),
            pltpu.VMEM((c, 1024), jnp.float32),
        ],
    )
    def k(x_hbm, i0_hbm, i1_hbm, out_hbm, i0_v, i1_v, rows_v):
        wid = lax.axis_index("s") * 2 + lax.axis_index("c")
        base = wid * RPW

        @pl.loop(0, RPW, step=c)
        def _(off):
            pltpu.sync_copy(i0_hbm.at[pl.ds(base + off, c)], i0_v)
            pltpu.sync_copy(i1_hbm.at[pl.ds(base + off, c)], i1_v)
            pltpu.sync_copy(x_hbm.at[i0_v], rows_v)
            pltpu.sync_copy(rows_v, out_hbm.at[i1_v])

    return k(x0s, dest0, dest1)


def _sc_gather_out(os_, dest1):
    @functools.partial(
        pl.kernel,
        out_type=jax.ShapeDtypeStruct((BT, 512), jnp.float32),
        mesh=_sc_mesh(),
        scratch_types=[
            pltpu.VMEM((RPW,), jnp.int32),
            pltpu.VMEM((RPW, 512), jnp.float32),
        ],
    )
    def k(x_hbm, idx_hbm, out_hbm, idx_v, rows_v):
        wid = lax.axis_index("s") * 2 + lax.axis_index("c")
        base = wid * RPW
        pltpu.sync_copy(idx_hbm.at[pl.ds(base, RPW)], idx_v)
        pltpu.sync_copy(x_hbm.at[idx_v], rows_v)
        pltpu.sync_copy(rows_v, out_hbm.at[pl.ds(base, RPW)])

    return k(os_, dest1)


def _gemm_a_body(gid_ref, x_ref, w1_ref, b1_ref, w2_ref, b2_ref, o_ref):
    g = gid_ref[pl.program_id(0)]
    x = x_ref[...]
    h = jnp.dot(x, w1_ref[g], preferred_element_type=jnp.float32)
    h = jnp.maximum(h + b1_ref[g], 0.0)
    o = jnp.dot(h, w2_ref[g], preferred_element_type=jnp.float32)
    o_ref[...] = jnp.maximum(o + b2_ref[g], 0.0)


def _gemm_a(gid, xs, w1, b1, w2, b2):
    grid_spec = pltpu.PrefetchScalarGridSpec(
        num_scalar_prefetch=1,
        grid=(NB,),
        in_specs=[
            pl.BlockSpec((T, 128), lambda b, g: (b, 0)),
            pl.BlockSpec((NM, 128, 1024), lambda b, g: (0, 0, 0)),
            pl.BlockSpec((NM, 1024), lambda b, g: (0, 0)),
            pl.BlockSpec((NM, 1024, 1024), lambda b, g: (0, 0, 0)),
            pl.BlockSpec((NM, 1024), lambda b, g: (0, 0)),
        ],
        out_specs=pl.BlockSpec((T, 1024), lambda b, g: (b, 0)),
    )
    return pl.pallas_call(
        _gemm_a_body,
        grid_spec=grid_spec,
        out_shape=jax.ShapeDtypeStruct((PAD, 1024), jnp.float32),
    )(gid, xs, w1, b1, w2, b2)


def _gemm_b_body(gid_ref, f1_ref, x0_ref, pw_ref, pb_ref, w1_ref, b1_ref,
                 w2_ref, b2_ref, o_ref):
    g = gid_ref[pl.program_id(0)]
    p = jnp.dot(f1_ref[...], pw_ref[g], preferred_element_type=jnp.float32)
    p = jnp.maximum(p + pb_ref[g], 0.0)
    w1 = w1_ref[0]
    h = jnp.dot(x0_ref[...], w1[:1024], preferred_element_type=jnp.float32)
    h = h + jnp.dot(p, w1[1024:], preferred_element_type=jnp.float32)
    h = jnp.maximum(h + b1_ref[g], 0.0)
    o = jnp.dot(h, w2_ref[g], preferred_element_type=jnp.float32)
    o_ref[...] = o + b2_ref[g]


def _gemm_b(gid, f1s, x0g, pw, pb, w1, b1, w2, b2):
    grid_spec = pltpu.PrefetchScalarGridSpec(
        num_scalar_prefetch=1,
        grid=(NB,),
        in_specs=[
            pl.BlockSpec((T, 128), lambda b, g: (b, 0)),
            pl.BlockSpec((T, 1024), lambda b, g: (b, 0)),
            pl.BlockSpec((NM, 128, 512), lambda b, g: (0, 0, 0)),
            pl.BlockSpec((NM, 512), lambda b, g: (0, 0)),
            pl.BlockSpec((1, 1536, 1024), lambda b, g: (g[b], 0, 0)),
            pl.BlockSpec((NM, 1024), lambda b, g: (0, 0)),
            pl.BlockSpec((NM, 1024, 512), lambda b, g: (0, 0, 0)),
            pl.BlockSpec((NM, 512), lambda b, g: (0, 0)),
        ],
        out_specs=pl.BlockSpec((T, 512), lambda b, g: (b, 0)),
    )
    return pl.pallas_call(
        _gemm_b_body,
        grid_spec=grid_spec,
        out_shape=jax.ShapeDtypeStruct((PAD, 512), jnp.float32),
    )(gid, f1s, x0g, pw, pb, w1, b1, w2, b2)


def kernel(input_val, n0_W1, n0_b1, n0_W2, n0_b2, n1_preW, n1_preb,
           n1_W1, n1_b1, n1_W2, n1_b2):
    feats0 = input_val[:, 0:128]
    feats1 = input_val[:, 128:256]
    oh = input_val[:, 256:272]

    dest0c, dest1c, gid0c, gid1c = _route(oh)
    dest0 = dest0c.reshape(BT)
    dest1 = dest1c.reshape(BT)
    gid0 = gid0c.reshape(NB)
    gid1 = gid1c.reshape(NB)

    xs0 = _sc_scatter_rows(feats0, dest0, 128)
    f1s = _sc_scatter_rows(feats1, dest1, 128)
    x0s = _gemm_a(gid0, xs0, n0_W1, n0_b1, n0_W2, n0_b2)
    x0g = _sc_transfer(x0s, dest0, dest1)
    os_ = _gemm_b(gid1, f1s, x0g, n1_preW, n1_preb, n1_W1, n1_b1, n1_W2, n1_b2)
    return _sc_gather_out(os_, dest1)
